# Initial kernel scaffold; baseline (speedup 1.0000x reference)
import jax
import jax.numpy as jnp
from jax.experimental import pallas as pl


def kernel(x, thresholds):
    raise NotImplementedError("write your pallas kernel here")


# speedup vs baseline: 1.4947x; 1.4947x over previous
import functools

import jax
import jax.numpy as jnp
from jax import lax
from jax.experimental import pallas as pl
from jax.experimental.pallas import tpu as pltpu, tpu_sc as plsc

_NW = 32
_C = 2048


@functools.lru_cache(maxsize=None)
def _build(n):
    per_w = n // _NW
    n_chunks = per_w // _C
    groups = _C // 16

    mesh = plsc.VectorSubcoreMesh(core_axis_name="c", subcore_axis_name="s")

    @functools.partial(
        pl.kernel,
        mesh=mesh,
        out_type=jax.ShapeDtypeStruct((n * 16,), jnp.float32),
        compiler_params=pltpu.CompilerParams(needs_layout_passes=False),
        scratch_types=[
            pltpu.VMEM((16,), jnp.float32),
            pltpu.VMEM((_C,), jnp.float32),
            pltpu.VMEM((_C,), jnp.float32),
            pltpu.VMEM((_C * 16,), jnp.float32),
            pltpu.VMEM((_C * 16,), jnp.float32),
            pltpu.SemaphoreType.DMA,
            pltpu.SemaphoreType.DMA,
        ],
    )
    def sc_kernel(x_hbm, thr_hbm, out_hbm, thr_v, x_v0, x_v1, o_v0, o_v1, s0, s1):
        wid = lax.axis_index("s") * 2 + lax.axis_index("c")
        base = wid * per_w
        pltpu.sync_copy(thr_hbm, thr_v)

        lane16 = lax.iota(jnp.int32, 16) * 16
        ones = jnp.ones((16,), jnp.float32)
        zeros = jnp.zeros((16,), jnp.float32)

        def do_chunk(ci, x_v, o_v, sem):
            off = base + ci * _C
            pltpu.sync_copy(x_hbm.at[pl.ds(off, _C)], x_v)

            @pl.when(ci >= 2)
            def _():
                pltpu.make_async_copy(
                    o_v, out_hbm.at[pl.ds((off - 2 * _C) * 16, _C * 16)], sem
                ).wait()

            def group(g, carry):
                xv = x_v[pl.ds(g * 16, 16)]
                pos = jnp.zeros((16,), jnp.int32)
                for s in (8, 4, 2, 1):
                    t = plsc.load_gather(thr_v, [pos + (s - 1)])
                    pos = pos + jnp.where(t < xv, s, 0)
                gbase = g * 256
                for r in range(16):
                    o_v[pl.ds(gbase + r * 16, 16)] = zeros
                plsc.store_scatter(o_v, [gbase + lane16 + pos], ones)
                return carry

            lax.fori_loop(0, groups, group, 0)
            pltpu.async_copy(o_v, out_hbm.at[pl.ds(off * 16, _C * 16)], sem)

        def outer(i, carry):
            do_chunk(2 * i, x_v0, o_v0, s0)
            do_chunk(2 * i + 1, x_v1, o_v1, s1)
            return carry

        lax.fori_loop(0, n_chunks // 2, outer, 0)

        pltpu.make_async_copy(
            o_v0, out_hbm.at[pl.ds((base + (n_chunks - 2) * _C) * 16, _C * 16)], s0
        ).wait()
        pltpu.make_async_copy(
            o_v1, out_hbm.at[pl.ds((base + (n_chunks - 1) * _C) * 16, _C * 16)], s1
        ).wait()

    return sc_kernel


def kernel(x, thresholds):
    n = x.shape[0]
    x_flat = x.reshape(n)
    thr_pad = jnp.concatenate(
        [thresholds.astype(jnp.float32), jnp.full((1,), jnp.inf, jnp.float32)]
    )
    out_flat = _build(n)(x_flat, thr_pad)
    return out_flat.reshape(n, 16)


# speedup vs baseline: 1.5282x; 1.0224x over previous
import functools

import jax
import jax.numpy as jnp
from jax import lax
from jax.experimental import pallas as pl
from jax.experimental.pallas import tpu as pltpu, tpu_sc as plsc

_NW = 32
_C = 2048
_U = 4


@functools.lru_cache(maxsize=None)
def _build(n):
    per_w = n // _NW
    n_chunks = per_w // _C
    groups = _C // 16

    mesh = plsc.VectorSubcoreMesh(core_axis_name="c", subcore_axis_name="s")

    @functools.partial(
        pl.kernel,
        mesh=mesh,
        out_type=jax.ShapeDtypeStruct((n * 16,), jnp.float32),
        compiler_params=pltpu.CompilerParams(needs_layout_passes=False),
        scratch_types=[
            pltpu.VMEM((16,), jnp.float32),
            pltpu.VMEM((_C,), jnp.float32),
            pltpu.VMEM((_C,), jnp.float32),
            pltpu.VMEM((_C * 16,), jnp.float32),
            pltpu.VMEM((_C * 16,), jnp.float32),
            pltpu.SemaphoreType.DMA,
            pltpu.SemaphoreType.DMA,
            pltpu.SemaphoreType.DMA,
            pltpu.SemaphoreType.DMA,
        ],
    )
    def sc_kernel(
        x_hbm, thr_hbm, out_hbm, thr_v, x_v0, x_v1, o_v0, o_v1, so0, so1, sx0, sx1
    ):
        wid = lax.axis_index("s") * 2 + lax.axis_index("c")
        base = wid * per_w
        pltpu.sync_copy(thr_hbm, thr_v)

        lane16 = lax.iota(jnp.int32, 16) * 16
        ones = jnp.ones((16,), jnp.float32)
        zeros = jnp.zeros((16,), jnp.float32)

        def x_slice(ci):
            return x_hbm.at[pl.ds(base + ci * _C, _C)]

        def o_slice(ci):
            return out_hbm.at[pl.ds((base + ci * _C) * 16, _C * 16)]

        def do_chunk(ci, x_v, o_v, so, x_nv, sx, sx_n):
            pltpu.make_async_copy(x_slice(ci), x_v, sx).wait()

            @pl.when(ci + 1 < n_chunks)
            def _():
                pltpu.async_copy(x_slice(ci + 1), x_nv, sx_n)

            @pl.when(ci >= 2)
            def _():
                pltpu.make_async_copy(o_v, o_slice(ci - 2), so).wait()

            def group(gi, carry):
                for u in range(_U):
                    g = gi * _U + u
                    xv = x_v[pl.ds(g * 16, 16)]
                    pos = jnp.zeros((16,), jnp.int32)
                    for s in (8, 4, 2, 1):
                        t = plsc.load_gather(thr_v, [pos + (s - 1)])
                        pos = pos + jnp.where(t < xv, s, 0)
                    gbase = g * 256
                    for r in range(16):
                        o_v[pl.ds(gbase + r * 16, 16)] = zeros
                    plsc.store_scatter(o_v, [gbase + lane16 + pos], ones)
                return carry

            lax.fori_loop(0, groups // _U, group, 0)
            pltpu.async_copy(o_v, o_slice(ci), so)

        pltpu.async_copy(x_slice(0), x_v0, sx0)

        def outer(i, carry):
            do_chunk(2 * i, x_v0, o_v0, so0, x_v1, sx0, sx1)
            do_chunk(2 * i + 1, x_v1, o_v1, so1, x_v0, sx1, sx0)
            return carry

        lax.fori_loop(0, n_chunks // 2, outer, 0)

        pltpu.make_async_copy(o_v0, o_slice(n_chunks - 2), so0).wait()
        pltpu.make_async_copy(o_v1, o_slice(n_chunks - 1), so1).wait()

    return sc_kernel


def kernel(x, thresholds):
    n = x.shape[0]
    x_flat = x.reshape(n)
    thr_pad = jnp.concatenate(
        [thresholds.astype(jnp.float32), jnp.full((1,), jnp.inf, jnp.float32)]
    )
    out_flat = _build(n)(x_flat, thr_pad)
    return out_flat.reshape(n, 16)


# speedup vs baseline: 1.6218x; 1.0612x over previous
import functools

import jax
import jax.numpy as jnp
from jax import lax
from jax.experimental import pallas as pl
from jax.experimental.pallas import tpu as pltpu, tpu_sc as plsc

_NW = 32
_C = 2048
_U = 4


@functools.lru_cache(maxsize=None)
def _build(n):
    per_w = n // _NW
    n_chunks = per_w // _C
    groups = _C // 16

    mesh = plsc.VectorSubcoreMesh(core_axis_name="c", subcore_axis_name="s")

    @functools.partial(
        pl.kernel,
        mesh=mesh,
        out_type=jax.ShapeDtypeStruct((n * 16,), jnp.float32),
        compiler_params=pltpu.CompilerParams(needs_layout_passes=False),
        scratch_types=[
            pltpu.VMEM((16 * 16,), jnp.float32),
            pltpu.VMEM((_C,), jnp.float32),
            pltpu.VMEM((_C,), jnp.float32),
            pltpu.VMEM((_C * 16,), jnp.float32),
            pltpu.VMEM((_C * 16,), jnp.float32),
            pltpu.SemaphoreType.DMA,
            pltpu.SemaphoreType.DMA,
            pltpu.SemaphoreType.DMA,
            pltpu.SemaphoreType.DMA,
        ],
    )
    def sc_kernel(
        x_hbm, thr_hbm, out_hbm, thr_v, x_v0, x_v1, o_v0, o_v1, so0, so1, sx0, sx1
    ):
        wid = lax.axis_index("s") * 2 + lax.axis_index("c")
        base = wid * per_w
        pltpu.sync_copy(thr_hbm, thr_v)

        lane16 = lax.iota(jnp.int32, 16) * 16
        ones = jnp.ones((16,), jnp.float32)
        zeros = jnp.zeros((16,), jnp.float32)
        one_i = jnp.ones((16,), jnp.int32)
        zero_i = jnp.zeros((16,), jnp.int32)
        tsplat = [thr_v[pl.ds(j * 16, 16)] for j in range(15)]

        def x_slice(ci):
            return x_hbm.at[pl.ds(base + ci * _C, _C)]

        def o_slice(ci):
            return out_hbm.at[pl.ds((base + ci * _C) * 16, _C * 16)]

        def do_chunk(ci, x_v, o_v, so, x_nv, sx, sx_n):
            pltpu.make_async_copy(x_slice(ci), x_v, sx).wait()

            @pl.when(ci + 1 < n_chunks)
            def _():
                pltpu.async_copy(x_slice(ci + 1), x_nv, sx_n)

            @pl.when(ci >= 2)
            def _():
                pltpu.make_async_copy(o_v, o_slice(ci - 2), so).wait()

            def group(gi, carry):
                for u in range(_U):
                    g = gi * _U + u
                    xv = x_v[pl.ds(g * 16, 16)]
                    terms = [jnp.where(t < xv, one_i, zero_i) for t in tsplat]
                    while len(terms) > 1:
                        terms = [
                            terms[k] + terms[k + 1] if k + 1 < len(terms) else terms[k]
                            for k in range(0, len(terms), 2)
                        ]
                    pos = terms[0]
                    gbase = g * 256
                    for r in range(16):
                        o_v[pl.ds(gbase + r * 16, 16)] = zeros
                    plsc.store_scatter(o_v, [gbase + lane16 + pos], ones)
                return carry

            lax.fori_loop(0, groups // _U, group, 0)
            pltpu.async_copy(o_v, o_slice(ci), so)

        pltpu.async_copy(x_slice(0), x_v0, sx0)

        def outer(i, carry):
            do_chunk(2 * i, x_v0, o_v0, so0, x_v1, sx0, sx1)
            do_chunk(2 * i + 1, x_v1, o_v1, so1, x_v0, sx1, sx0)
            return carry

        lax.fori_loop(0, n_chunks // 2, outer, 0)

        pltpu.make_async_copy(o_v0, o_slice(n_chunks - 2), so0).wait()
        pltpu.make_async_copy(o_v1, o_slice(n_chunks - 1), so1).wait()

    return sc_kernel


def kernel(x, thresholds):
    n = x.shape[0]
    x_flat = x.reshape(n)
    thr_rep = jnp.concatenate(
        [
            jnp.repeat(thresholds.astype(jnp.float32), 16),
            jnp.full((16,), jnp.inf, jnp.float32),
        ]
    )
    out_flat = _build(n)(x_flat, thr_rep)
    return out_flat.reshape(n, 16)


# speedup vs baseline: 16.1421x; 9.9535x over previous
import functools

import jax
import jax.numpy as jnp
from jax import lax
from jax.experimental import pallas as pl
from jax.experimental.pallas import tpu as pltpu, tpu_sc as plsc

_NW = 32
_C = 2048
_U = 4


@functools.lru_cache(maxsize=None)
def _build(n):
    per_w = n // _NW
    n_chunks = per_w // _C
    groups = _C // 16
    plane = n * 8
    cplane = _C * 8

    mesh = plsc.VectorSubcoreMesh(core_axis_name="c", subcore_axis_name="s")

    @functools.partial(
        pl.kernel,
        mesh=mesh,
        out_type=jax.ShapeDtypeStruct((n * 16,), jnp.float32),
        compiler_params=pltpu.CompilerParams(needs_layout_passes=False),
        scratch_types=[
            pltpu.VMEM((16 * 16,), jnp.float32),
            pltpu.VMEM((_C,), jnp.float32),
            pltpu.VMEM((_C,), jnp.float32),
            pltpu.VMEM((_C * 16,), jnp.float32),
            pltpu.VMEM((_C * 16,), jnp.float32),
            pltpu.SemaphoreType.DMA,
            pltpu.SemaphoreType.DMA,
            pltpu.SemaphoreType.DMA,
            pltpu.SemaphoreType.DMA,
        ],
    )
    def sc_kernel(
        x_hbm, thr_hbm, out_hbm, thr_v, x_v0, x_v1, o_v0, o_v1, so0, so1, sx0, sx1
    ):
        wid = lax.axis_index("s") * 2 + lax.axis_index("c")
        base = wid * per_w
        pltpu.sync_copy(thr_hbm, thr_v)

        ones = jnp.ones((16,), jnp.float32)
        tsplat = [thr_v[pl.ds(j * 16, 16)] for j in range(15)]

        def x_slice(ci):
            return x_hbm.at[pl.ds(base + ci * _C, _C)]

        def do_chunk(ci, x_v, o_v, so, x_nv, sx, sx_n):
            pltpu.make_async_copy(x_slice(ci), x_v, sx).wait()

            @pl.when(ci + 1 < n_chunks)
            def _():
                pltpu.async_copy(x_slice(ci + 1), x_nv, sx_n)

            @pl.when(ci >= 2)
            def _():
                off8 = (base + (ci - 2) * _C) * 8
                pltpu.make_async_copy(
                    o_v.at[pl.ds(0, cplane)], out_hbm.at[pl.ds(off8, cplane)], so
                ).wait()
                pltpu.make_async_copy(
                    o_v.at[pl.ds(cplane, cplane)],
                    out_hbm.at[pl.ds(plane + off8, cplane)],
                    so,
                ).wait()

            def group(gi, carry):
                for u in range(_U):
                    g = gi * _U + u
                    xv = x_v[pl.ds(g * 16, 16)]
                    b = [jnp.where(t < xv, 1.0, 0.0) for t in tsplat]
                    cols = (
                        [ones - b[0]]
                        + [b[j - 1] - b[j] for j in range(1, 15)]
                        + [b[14]]
                    )
                    gbase = (g // 8) * 1024 + (g % 8) * 16
                    for j in range(16):
                        addr = (j // 8) * cplane + (j % 8) * 128 + gbase
                        o_v[pl.ds(addr, 16)] = cols[j]
                return carry

            lax.fori_loop(0, groups // _U, group, 0)
            off8 = (base + ci * _C) * 8
            pltpu.async_copy(
                o_v.at[pl.ds(0, cplane)], out_hbm.at[pl.ds(off8, cplane)], so
            )
            pltpu.async_copy(
                o_v.at[pl.ds(cplane, cplane)],
                out_hbm.at[pl.ds(plane + off8, cplane)],
                so,
            )

        pltpu.async_copy(x_slice(0), x_v0, sx0)

        def outer(i, carry):
            do_chunk(2 * i, x_v0, o_v0, so0, x_v1, sx0, sx1)
            do_chunk(2 * i + 1, x_v1, o_v1, so1, x_v0, sx1, sx0)
            return carry

        lax.fori_loop(0, n_chunks // 2, outer, 0)

        for o_v, so, ci in ((o_v0, so0, n_chunks - 2), (o_v1, so1, n_chunks - 1)):
            off8 = (base + ci * _C) * 8
            pltpu.make_async_copy(
                o_v.at[pl.ds(0, cplane)], out_hbm.at[pl.ds(off8, cplane)], so
            ).wait()
            pltpu.make_async_copy(
                o_v.at[pl.ds(cplane, cplane)],
                out_hbm.at[pl.ds(plane + off8, cplane)],
                so,
            ).wait()

    return sc_kernel


def kernel(x, thresholds):
    n = x.shape[0]
    x_flat = x.reshape(n)
    thr_rep = jnp.concatenate(
        [
            jnp.repeat(thresholds.astype(jnp.float32), 16),
            jnp.full((16,), jnp.inf, jnp.float32),
        ]
    )
    out_lin = _build(n)(x_flat, thr_rep)
    out = (
        out_lin.reshape(2, n // 128, 8, 128)
        .transpose(1, 3, 0, 2)
        .reshape(n, 16)
    )
    return out


# speedup vs baseline: 16.2002x; 1.0036x over previous
import functools

import jax
import jax.numpy as jnp
from jax import lax
from jax.experimental import pallas as pl
from jax.experimental.pallas import tpu as pltpu, tpu_sc as plsc

_NW = 32
_C = 2048
_U = 4


@functools.lru_cache(maxsize=None)
def _build(n):
    per_w = n // _NW
    n_chunks = per_w // _C
    groups = _C // 16
    plane = n * 8
    cplane = _C * 8

    mesh = plsc.VectorSubcoreMesh(core_axis_name="c", subcore_axis_name="s")

    @functools.partial(
        pl.kernel,
        mesh=mesh,
        out_type=jax.ShapeDtypeStruct((n * 16,), jnp.float32),
        compiler_params=pltpu.CompilerParams(needs_layout_passes=False),
        scratch_types=[
            pltpu.VMEM((15,), jnp.float32),
            pltpu.VMEM((_C,), jnp.float32),
            pltpu.VMEM((_C,), jnp.float32),
            pltpu.VMEM((_C * 16,), jnp.float32),
            pltpu.VMEM((_C * 16,), jnp.float32),
            pltpu.SemaphoreType.DMA,
            pltpu.SemaphoreType.DMA,
            pltpu.SemaphoreType.DMA,
            pltpu.SemaphoreType.DMA,
        ],
    )
    def sc_kernel(
        x_hbm, thr_hbm, out_hbm, thr_v, x_v0, x_v1, o_v0, o_v1, so0, so1, sx0, sx1
    ):
        wid = lax.axis_index("s") * 2 + lax.axis_index("c")
        base = wid * per_w
        pltpu.sync_copy(thr_hbm, thr_v)

        ones = jnp.ones((16,), jnp.float32)
        tsplat = [
            plsc.load_gather(thr_v, [jnp.full((16,), j, jnp.int32)])
            for j in range(15)
        ]

        def x_slice(ci):
            return x_hbm.at[pl.ds(base + ci * _C, _C)]

        def do_chunk(ci, x_v, o_v, so, x_nv, sx, sx_n):
            pltpu.make_async_copy(x_slice(ci), x_v, sx).wait()

            @pl.when(ci + 1 < n_chunks)
            def _():
                pltpu.async_copy(x_slice(ci + 1), x_nv, sx_n)

            @pl.when(ci >= 2)
            def _():
                off8 = (base + (ci - 2) * _C) * 8
                pltpu.make_async_copy(
                    o_v.at[pl.ds(0, cplane)], out_hbm.at[pl.ds(off8, cplane)], so
                ).wait()
                pltpu.make_async_copy(
                    o_v.at[pl.ds(cplane, cplane)],
                    out_hbm.at[pl.ds(plane + off8, cplane)],
                    so,
                ).wait()

            def group(gi, carry):
                for u in range(_U):
                    g = gi * _U + u
                    xv = x_v[pl.ds(g * 16, 16)]
                    b = [jnp.where(t < xv, 1.0, 0.0) for t in tsplat]
                    cols = (
                        [ones - b[0]]
                        + [b[j - 1] - b[j] for j in range(1, 15)]
                        + [b[14]]
                    )
                    gbase = (g // 8) * 1024 + (g % 8) * 16
                    for j in range(16):
                        addr = (j // 8) * cplane + (j % 8) * 128 + gbase
                        o_v[pl.ds(addr, 16)] = cols[j]
                return carry

            lax.fori_loop(0, groups // _U, group, 0)
            off8 = (base + ci * _C) * 8
            pltpu.async_copy(
                o_v.at[pl.ds(0, cplane)], out_hbm.at[pl.ds(off8, cplane)], so
            )
            pltpu.async_copy(
                o_v.at[pl.ds(cplane, cplane)],
                out_hbm.at[pl.ds(plane + off8, cplane)],
                so,
            )

        pltpu.async_copy(x_slice(0), x_v0, sx0)

        def outer(i, carry):
            do_chunk(2 * i, x_v0, o_v0, so0, x_v1, sx0, sx1)
            do_chunk(2 * i + 1, x_v1, o_v1, so1, x_v0, sx1, sx0)
            return carry

        lax.fori_loop(0, n_chunks // 2, outer, 0)

        for o_v, so, ci in ((o_v0, so0, n_chunks - 2), (o_v1, so1, n_chunks - 1)):
            off8 = (base + ci * _C) * 8
            pltpu.make_async_copy(
                o_v.at[pl.ds(0, cplane)], out_hbm.at[pl.ds(off8, cplane)], so
            ).wait()
            pltpu.make_async_copy(
                o_v.at[pl.ds(cplane, cplane)],
                out_hbm.at[pl.ds(plane + off8, cplane)],
                so,
            ).wait()

    return sc_kernel


def kernel(x, thresholds):
    n = x.shape[0]
    x_flat = x.reshape(n)
    out_lin = _build(n)(x_flat, thresholds)
    out = (
        out_lin.reshape(2, n // 128, 8, 128)
        .transpose(1, 3, 0, 2)
        .reshape(n, 16)
    )
    return out
